# baseline (speedup 1.0000x reference)
import functools

import jax
import jax.numpy as jnp
from jax import lax
from jax.experimental import pallas as pl
from jax.experimental.pallas import tpu as pltpu
from jax.experimental.pallas import tpu_sc as plsc

_VOCAB = 100000
_EMBED = 128
_NCLASS = 4
_B = 4096
_HIST = 50
_PADC = 16

_NW = 32
_BAGS_W = _B // _NW
_TOK_W = _BAGS_W * _HIST
_CHUNK = 2 * _HIST
_NCHUNK = _TOK_W // _CHUNK
_NBUF = 4

_ROWS_BLK = 25000


def _proj_body(emb_ref, fct_ref, out_ref):
    out_ref[...] = jnp.dot(emb_ref[...], fct_ref[...],
                           preferred_element_type=jnp.float32)


def _project_table(emb_weight, fct_pad):
    return pl.pallas_call(
        _proj_body,
        grid=(_VOCAB // _ROWS_BLK,),
        in_specs=[
            pl.BlockSpec((_ROWS_BLK, _EMBED), lambda i: (i, 0)),
            pl.BlockSpec((_EMBED, _PADC), lambda i: (0, 0)),
        ],
        out_specs=pl.BlockSpec((_ROWS_BLK, _PADC), lambda i: (i, 0)),
        out_shape=jax.ShapeDtypeStruct((_VOCAB, _PADC), jnp.float32),
    )(emb_weight, fct_pad)


def _sc_bag_mean(text3, proj):
    mesh = plsc.VectorSubcoreMesh(core_axis_name="c", subcore_axis_name="s")

    @functools.partial(
        pl.kernel,
        mesh=mesh,
        compiler_params=pltpu.CompilerParams(use_tc_tiling_on_sc=False),
        out_type=jax.ShapeDtypeStruct((_NW, _BAGS_W, _PADC), jnp.float32),
        scratch_types=[
            pltpu.VMEM((_NCHUNK, _CHUNK), jnp.int32),
            pltpu.VMEM((_NBUF, _CHUNK, _PADC), jnp.float32),
            pltpu.VMEM((_BAGS_W, _PADC), jnp.float32),
        ] + [pltpu.SemaphoreType.DMA] * _NBUF,
    )
    def sc_fn(text_hbm, proj_hbm, out_hbm, tok_v, rows_v, out_v, *sems):
        wid = lax.axis_index("s") * 2 + lax.axis_index("c")
        pltpu.sync_copy(text_hbm.at[wid], tok_v)

        def copy(q, k):
            return pltpu.make_async_copy(
                proj_hbm.at[tok_v.at[q]], rows_v.at[k], sems[k])

        def reduce2(j, k):
            for h in range(2):
                base = h * _HIST
                accs = [rows_v[k, base + a] for a in range(4)]
                for t in range(4, _HIST):
                    accs[t % 4] = accs[t % 4] + rows_v[k, base + t]
                out_v[2 * j + h] = (accs[0] + accs[1]) + (accs[2] + accs[3])

        for k in range(_NBUF):
            copy(k, k).start()

        def group(g, _):
            for k in range(_NBUF):
                j = g * _NBUF + k
                copy(j, k).wait()
                copy(j + _NBUF, k).start()
                reduce2(j, k)
            return 0

        lax.fori_loop(0, _NCHUNK // _NBUF - 1, group, 0)

        for k in range(_NBUF):
            j = _NCHUNK - _NBUF + k
            copy(j, k).wait()
            reduce2(j, k)

        pltpu.sync_copy(out_v, out_hbm.at[wid])

    return sc_fn(text3, proj)


def kernel(text, offsets, emb_weight, fc_weight, fc_bias):
    del offsets
    fct_pad = jnp.zeros((_EMBED, _PADC), jnp.float32)
    fct_pad = fct_pad.at[:, :_NCLASS].set(fc_weight.T * jnp.float32(1.0 / _HIST))
    text3 = text.astype(jnp.int32).reshape(_NW, _NCHUNK, _CHUNK)
    proj = _project_table(emb_weight, fct_pad)
    out = _sc_bag_mean(text3, proj)
    return out.reshape(_B, _PADC)[:, :_NCLASS] + fc_bias[None, :]


# speedup vs baseline: 1.6026x; 1.6026x over previous
import functools

import jax
import jax.numpy as jnp
from jax import lax
from jax.experimental import pallas as pl
from jax.experimental.pallas import tpu as pltpu
from jax.experimental.pallas import tpu_sc as plsc

_VOCAB = 100000
_EMBED = 128
_NCLASS = 4
_B = 4096
_HIST = 50
_LANE = 16

_NW = 32
_BAGS_W = _B // _NW
_TOK_W = _BAGS_W * _HIST
_CHUNK = 2 * _HIST
_NCHUNK = _TOK_W // _CHUNK
_NBUF = 4
_NSEG = _EMBED // _LANE


def _take16(x, idx):
    return lax.gather(
        x, idx[:, None],
        dimension_numbers=lax.GatherDimensionNumbers(
            offset_dims=(), collapsed_slice_dims=(0,), start_index_map=(0,)),
        slice_sizes=(1,),
        mode=lax.GatherScatterMode.PROMISE_IN_BOUNDS)


def _sc_bag_logits(text3, emb_weight, fcs):
    mesh = plsc.VectorSubcoreMesh(core_axis_name="c", subcore_axis_name="s")

    @functools.partial(
        pl.kernel,
        mesh=mesh,
        compiler_params=pltpu.CompilerParams(use_tc_tiling_on_sc=False),
        out_type=jax.ShapeDtypeStruct((_NW, _BAGS_W, _LANE), jnp.float32),
        scratch_types=[
            pltpu.VMEM((_NCHUNK, _CHUNK), jnp.int32),
            pltpu.VMEM((_NBUF, _CHUNK, _EMBED), jnp.float32),
            pltpu.VMEM((_NCLASS, _EMBED), jnp.float32),
            pltpu.VMEM((_BAGS_W, _LANE), jnp.float32),
        ] + [pltpu.SemaphoreType.DMA] * _NBUF,
    )
    def sc_fn(text_hbm, emb_hbm, fcs_hbm, out_hbm,
              tok_v, rows_v, fcs_v, out_v, *sems):
        wid = lax.axis_index("s") * 2 + lax.axis_index("c")
        pltpu.sync_copy(text_hbm.at[wid], tok_v)
        pltpu.sync_copy(fcs_hbm, fcs_v)

        def copy(q, k):
            return pltpu.make_async_copy(
                emb_hbm.at[tok_v.at[q]], rows_v.at[k], sems[k])

        def reduce2(j, k):
            for h in range(2):
                base = h * _HIST

                def tok_body(t, accs):
                    return tuple(
                        accs[r] + rows_v[k, base + t, pl.ds(r * _LANE, _LANE)]
                        for r in range(_NSEG))

                first = tuple(rows_v[k, base, pl.ds(r * _LANE, _LANE)]
                              for r in range(_NSEG))
                accs = lax.fori_loop(1, _HIST, tok_body, first)
                lane = lax.iota(jnp.int32, _LANE)
                vec = jnp.zeros((_LANE,), jnp.float32)
                for c in range(_NCLASS):
                    w = [fcs_v[c, pl.ds(r * _LANE, _LANE)]
                         for r in range(_NSEG)]
                    p = accs[0] * w[0]
                    for r in range(1, _NSEG):
                        p = p + accs[r] * w[r]
                    for sh in (8, 4, 2, 1):
                        perm = (lane + sh) % _LANE
                        p = p + _take16(p, perm)
                    vec = jnp.where(lane == c, p, vec)
                out_v[2 * j + h] = vec

        for k in range(_NBUF):
            copy(k, k).start()

        def group(g, _):
            for k in range(_NBUF):
                j = g * _NBUF + k
                copy(j, k).wait()
                copy(j + _NBUF, k).start()
                reduce2(j, k)
            return 0

        lax.fori_loop(0, _NCHUNK // _NBUF - 1, group, 0)

        for k in range(_NBUF):
            j = _NCHUNK - _NBUF + k
            copy(j, k).wait()
            reduce2(j, k)

        pltpu.sync_copy(out_v, out_hbm.at[wid])

    return sc_fn(text3, emb_weight, fcs)


def kernel(text, offsets, emb_weight, fc_weight, fc_bias):
    del offsets
    fcs = fc_weight * jnp.float32(1.0 / _HIST)
    text3 = text.astype(jnp.int32).reshape(_NW, _NCHUNK, _CHUNK)
    out = _sc_bag_logits(text3, emb_weight, fcs)
    return out.reshape(_B, _LANE)[:, :_NCLASS] + fc_bias[None, :]
